# Initial kernel scaffold; baseline (speedup 1.0000x reference)
import jax
import jax.numpy as jnp
from jax.experimental import pallas as pl


def kernel(gate_inputs, inputs, Wg, bg, W1, b1, W2, b2):
    raise NotImplementedError("write your pallas kernel here")


# speedup vs baseline: 1.8091x; 1.8091x over previous
import functools

import jax
import jax.numpy as jnp
from jax import lax
from jax.experimental import pallas as pl
from jax.experimental.pallas import tpu as pltpu
from jax.experimental.pallas import tpu_sc as plsc

_E = 8
_D = 1024
_FF = 4096
_T = 8192
_TM = 256
_G = 2 * _T
_GP = _G + _E * _TM
_NT = _GP // _TM
_NBLK = _T // _TM
_NW = 32
_CHW = _GP // _NW
_CHC = _GP // 16


def _route_body(wg_ref, bg_ref, x_ref, ri_ref, rw_ref, aux_ref, carry_ref):
    b = pl.program_id(0)
    lt = lax.dot_general(wg_ref[...], x_ref[...], (((0,), (1,)), ((), ())),
                         preferred_element_type=jnp.float32)
    lt = lt + bg_ref[...]
    eio = lax.broadcasted_iota(jnp.int32, (128, _TM), 0)
    valid = eio < _E
    neg = jnp.float32(-1e30)
    l0 = jnp.where(valid, lt, neg)
    m0 = jnp.max(l0, axis=0, keepdims=True)
    e0 = jnp.min(jnp.where((l0 == m0) & valid, eio, 127), axis=0, keepdims=True)
    l1 = jnp.where(eio == e0, neg, l0)
    m1 = jnp.max(l1, axis=0, keepdims=True)
    e1 = jnp.min(jnp.where((l1 == m1) & valid, eio, 127), axis=0, keepdims=True)
    t = jnp.exp(m1 - m0)
    inv = 1.0 / (1.0 + t)
    w0, w1 = inv, t * inv

    oh0 = (eio == e0)[:_E, :]
    oh1 = (eio == e1)[:_E, :]
    cnt = oh0.astype(jnp.int32) + oh1.astype(jnp.int32)
    incl = cnt
    for s in (1, 2, 4, 8, 16, 32, 64, 128):
        incl = incl + jnp.concatenate(
            [jnp.zeros((_E, s), jnp.int32), incl[:, :-s]], axis=1)
    excl = incl - cnt
    carry = carry_ref[:, 0:1]
    carry = jnp.where(b == 0, jnp.zeros_like(carry), carry)
    tot = excl + carry
    r0 = jnp.sum(jnp.where(oh0, tot, 0), axis=0, keepdims=True)
    r1 = jnp.sum(jnp.where(oh1, tot, 0), axis=0, keepdims=True)
    newc = carry + incl[:, _TM - 1:_TM]
    carry_ref[:, 0:1] = newc

    ri_ref[...] = jnp.concatenate(
        [e0, e1, r0, r1, jnp.zeros((4, _TM), jnp.int32)], axis=0)[None]
    rw_ref[...] = jnp.concatenate(
        [w0, w1, jnp.zeros((6, _TM), jnp.float32)], axis=0)[None]

    @pl.when(b == pl.num_programs(0) - 1)
    def _():
        pc = ((newc + (_TM - 1)) // _TM) * _TM
        cum = pc
        for s in (1, 2, 4):
            cum = cum + jnp.concatenate(
                [jnp.zeros((s, 1), jnp.int32), cum[:-s]], axis=0)
        gi = lax.broadcasted_iota(jnp.int32, (_E, 128), 1) * _TM
        te = jnp.minimum(jnp.sum((gi >= cum).astype(jnp.int32),
                                 axis=0, keepdims=True), _E - 1)
        lio = lax.broadcasted_iota(jnp.int32, (_E, 128), 1)
        sio = lax.broadcasted_iota(jnp.int32, (_E, 128), 0)
        offs = jnp.sum(jnp.where(sio < lio, jnp.broadcast_to(pc, (_E, 128)), 0),
                       axis=0, keepdims=True)
        aux_ref[...] = jnp.concatenate(
            [te, offs, jnp.zeros((6, 128), jnp.int32)], axis=0)


def _route_call(wg_pad, bg_col, gate_inputs):
    return pl.pallas_call(
        _route_body,
        grid=(_NBLK,),
        in_specs=[
            pl.BlockSpec((_D, 128), lambda b: (0, 0)),
            pl.BlockSpec((128, 1), lambda b: (0, 0)),
            pl.BlockSpec((_TM, _D), lambda b: (b, 0)),
        ],
        out_specs=[
            pl.BlockSpec((1, _E, _TM), lambda b: (b, 0, 0)),
            pl.BlockSpec((1, _E, _TM), lambda b: (b, 0, 0)),
            pl.BlockSpec((_E, 128), lambda b: (0, 0)),
        ],
        out_shape=[
            jax.ShapeDtypeStruct((_NBLK, _E, _TM), jnp.int32),
            jax.ShapeDtypeStruct((_NBLK, _E, _TM), jnp.float32),
            jax.ShapeDtypeStruct((_E, 128), jnp.int32),
        ],
        scratch_shapes=[pltpu.VMEM((_E, 128), jnp.int32)],
        compiler_params=pltpu.CompilerParams(
            dimension_semantics=("arbitrary",)),
    )(wg_pad, bg_col, gate_inputs)


def _pos_body(offs_ref, ri_ref, pp_ref):
    e0 = ri_ref[0, 0:1, :]
    e1 = ri_ref[0, 1:2, :]
    p0 = ri_ref[0, 2:3, :]
    p1 = ri_ref[0, 3:4, :]
    for e in range(_E):
        p0 = p0 + jnp.where(e0 == e, offs_ref[e], 0)
        p1 = p1 + jnp.where(e1 == e, offs_ref[e], 0)
    pp_ref[...] = jnp.concatenate(
        [p0, p1, jnp.zeros((6, _TM), jnp.int32)], axis=0)[None]


def _pos_call(offs16, ri):
    grid_spec = pltpu.PrefetchScalarGridSpec(
        num_scalar_prefetch=1,
        grid=(_NBLK,),
        in_specs=[pl.BlockSpec((1, _E, _TM), lambda b, o: (b, 0, 0))],
        out_specs=pl.BlockSpec((1, _E, _TM), lambda b, o: (b, 0, 0)),
    )
    return pl.pallas_call(
        _pos_body,
        grid_spec=grid_spec,
        out_shape=jax.ShapeDtypeStruct((_NBLK, _E, _TM), jnp.int32),
        compiler_params=pltpu.CompilerParams(
            dimension_semantics=("arbitrary",)),
    )(offs16, ri)


def _b1_body(pp_hbm, rw_hbm, zi_hbm, zf_hbm,
             tokp0_hbm, tokp1_hbm, wp0_hbm, wp1_hbm, pos0_hbm, pos1_hbm,
             pp_v, rw_v, pos_st, val_st, w_st,
             tokbuf, wbuf):
    c = lax.axis_index("c")
    s = lax.axis_index("s")
    wid = s * 2 + c
    base = wid * _TM
    pltpu.sync_copy(pp_hbm.at[wid], pp_v)
    pltpu.sync_copy(rw_hbm.at[wid], rw_v)
    pltpu.sync_copy(zi_hbm, tokbuf.at[pl.ds(s * _CHC, _CHC)])
    pltpu.sync_copy(zf_hbm, wbuf.at[pl.ds(s * _CHC, _CHC)])
    for i in range(16):
        sl = pl.ds(i * 16, 16)
        row, col = i // 8, (i % 8) * 16
        csl = pl.ds(col, 16)
        tk = lax.iota(jnp.int32, 16) + (base + i * 16 + 1)
        pos_st[row, csl] = pp_v[0, sl]
        pos_st[2 + row, csl] = pp_v[1, sl]
        val_st[row, csl] = tk
        val_st[2 + row, csl] = tk
        w_st[row, csl] = rw_v[0, sl]
        w_st[2 + row, csl] = rw_v[1, sl]
    plsc.subcore_barrier()
    for j in range(4):
        pltpu.sync_copy(val_st.at[j], tokbuf.at[pos_st.at[j]])
        pltpu.sync_copy(w_st.at[j], wbuf.at[pos_st.at[j]])
    plsc.subcore_barrier()
    csl = pl.ds(s * _CHC, _CHC)

    @pl.when(c == 0)
    def _():
        pltpu.sync_copy(tokbuf.at[csl], tokp0_hbm.at[csl])
        pltpu.sync_copy(wbuf.at[csl], wp0_hbm.at[csl])

    @pl.when(c == 1)
    def _():
        pltpu.sync_copy(tokbuf.at[csl], tokp1_hbm.at[csl])
        pltpu.sync_copy(wbuf.at[csl], wp1_hbm.at[csl])

    pltpu.sync_copy(pp_v.at[0], pos0_hbm.at[pl.ds(base, _TM)])
    pltpu.sync_copy(pp_v.at[1], pos1_hbm.at[pl.ds(base, _TM)])


def _b1_call(pp, rw, zi, zf):
    mesh = plsc.VectorSubcoreMesh(core_axis_name="c", subcore_axis_name="s")
    f = functools.partial(
        pl.kernel, mesh=mesh,
        out_type=[
            jax.ShapeDtypeStruct((_GP,), jnp.int32),
            jax.ShapeDtypeStruct((_GP,), jnp.int32),
            jax.ShapeDtypeStruct((_GP,), jnp.float32),
            jax.ShapeDtypeStruct((_GP,), jnp.float32),
            jax.ShapeDtypeStruct((_T,), jnp.int32),
            jax.ShapeDtypeStruct((_T,), jnp.int32),
        ],
        scratch_types=[
            pltpu.VMEM((_E, _TM), jnp.int32),
            pltpu.VMEM((_E, _TM), jnp.float32),
            pltpu.VMEM((4, 128), jnp.int32),
            pltpu.VMEM((4, 128), jnp.int32),
            pltpu.VMEM((4, 128), jnp.float32),
            pltpu.VMEM_SHARED((_GP,), jnp.int32),
            pltpu.VMEM_SHARED((_GP,), jnp.float32),
        ],
    )(_b1_body)
    return f(pp, rw, zi, zf)


def _b2_body(tokp0_hbm, tokp1_hbm, wp0_hbm, wp1_hbm, x_hbm, xs_hbm, ws_hbm,
             ta_v, tb_v, tok_v, wa_v, wb_v, wc_v, rows_v, sem):
    c = lax.axis_index("c")
    s = lax.axis_index("s")
    wid = s * 2 + c
    base = wid * _CHW
    sl = pl.ds(base, _CHW)
    pltpu.sync_copy(tokp0_hbm.at[sl], ta_v)
    pltpu.sync_copy(tokp1_hbm.at[sl], tb_v)
    pltpu.sync_copy(wp0_hbm.at[sl], wa_v)
    pltpu.sync_copy(wp1_hbm.at[sl], wb_v)
    for j in range(_CHW // 16):
        vs = pl.ds(j * 16, 16)
        tok_v[vs] = jnp.maximum(ta_v[vs] + tb_v[vs] - 1, 0)
        wc_v[vs] = wa_v[vs] + wb_v[vs]
    pltpu.sync_copy(wc_v, ws_hbm.at[sl])
    for k in range(_CHW // 64):
        pltpu.async_copy(x_hbm.at[tok_v.at[pl.ds(k * 64, 64)]],
                         rows_v, sem).wait()
        pltpu.sync_copy(rows_v, xs_hbm.at[pl.ds(base + k * 64, 64)])


def _b2_call(tokp0, tokp1, wp0, wp1, x):
    mesh = plsc.VectorSubcoreMesh(core_axis_name="c", subcore_axis_name="s")
    f = functools.partial(
        pl.kernel, mesh=mesh,
        out_type=[
            jax.ShapeDtypeStruct((_GP, _D), jnp.float32),
            jax.ShapeDtypeStruct((_GP,), jnp.float32),
        ],
        scratch_types=[
            pltpu.VMEM((_CHW,), jnp.int32),
            pltpu.VMEM((_CHW,), jnp.int32),
            pltpu.VMEM((_CHW,), jnp.int32),
            pltpu.VMEM((_CHW,), jnp.float32),
            pltpu.VMEM((_CHW,), jnp.float32),
            pltpu.VMEM((_CHW,), jnp.float32),
            pltpu.VMEM((64, _D), jnp.float32),
            pltpu.SemaphoreType.DMA,
        ],
    )(_b2_body)
    return f(tokp0, tokp1, wp0, wp1, x)


def _ffn_body(te_ref, x_ref, w1_ref, b1_ref, w2_ref, b2_ref, ws_ref, y_ref):
    h = jnp.dot(x_ref[...], w1_ref[0], preferred_element_type=jnp.float32)
    h = h + b1_ref[0]
    h = h * (1.0 / (1.0 + jnp.exp(-h)))
    y = jnp.dot(h, w2_ref[0], preferred_element_type=jnp.float32)
    y = y + b2_ref[0]
    y_ref[...] = y * ws_ref[...]


def _ffn_call(te, xs, w1, b1, w2, b2, ws):
    grid_spec = pltpu.PrefetchScalarGridSpec(
        num_scalar_prefetch=1,
        grid=(_NT,),
        in_specs=[
            pl.BlockSpec((_TM, _D), lambda g, te: (g, 0)),
            pl.BlockSpec((1, _D, _FF), lambda g, te: (te[g], 0, 0),
                         pipeline_mode=pl.Buffered(buffer_count=1)),
            pl.BlockSpec((1, 1, _FF), lambda g, te: (te[g], 0, 0)),
            pl.BlockSpec((1, _FF, _D), lambda g, te: (te[g], 0, 0),
                         pipeline_mode=pl.Buffered(buffer_count=1)),
            pl.BlockSpec((1, 1, _D), lambda g, te: (te[g], 0, 0)),
            pl.BlockSpec((_TM, 1), lambda g, te: (g, 0)),
        ],
        out_specs=pl.BlockSpec((_TM, _D), lambda g, te: (g, 0)),
    )
    return pl.pallas_call(
        _ffn_body,
        grid_spec=grid_spec,
        out_shape=jax.ShapeDtypeStruct((_GP, _D), jnp.float32),
        compiler_params=pltpu.CompilerParams(
            dimension_semantics=("arbitrary",),
            vmem_limit_bytes=128 * 1024 * 1024),
    )(te, xs, w1, b1, w2, b2, ws)


def _comb_body(y_hbm, pos0_hbm, pos1_hbm, out_hbm, p0_v, p1_v, a_v, b_v,
               sem0, sem1):
    c = lax.axis_index("c")
    s = lax.axis_index("s")
    wid = s * 2 + c
    base = wid * _TM
    pltpu.sync_copy(pos0_hbm.at[pl.ds(base, _TM)], p0_v)
    pltpu.sync_copy(pos1_hbm.at[pl.ds(base, _TM)], p1_v)
    for m in range(_TM // 32):
        ca = pltpu.async_copy(y_hbm.at[p0_v.at[pl.ds(m * 32, 32)]], a_v, sem0)
        cb = pltpu.async_copy(y_hbm.at[p1_v.at[pl.ds(m * 32, 32)]], b_v, sem1)
        ca.wait()
        cb.wait()

        def _add_row(r, carry):
            for cc in range(_D // 16):
                vs = pl.ds(cc * 16, 16)
                a_v[r, vs] = a_v[r, vs] + b_v[r, vs]
            return carry

        lax.fori_loop(0, 32, _add_row, 0)
        pltpu.sync_copy(a_v, out_hbm.at[pl.ds(base + m * 32, 32)])


def _comb_call(ys, pos0, pos1):
    mesh = plsc.VectorSubcoreMesh(core_axis_name="c", subcore_axis_name="s")
    f = functools.partial(
        pl.kernel, mesh=mesh,
        out_type=jax.ShapeDtypeStruct((_T, _D), jnp.float32),
        scratch_types=[
            pltpu.VMEM((_TM,), jnp.int32),
            pltpu.VMEM((_TM,), jnp.int32),
            pltpu.VMEM((32, _D), jnp.float32),
            pltpu.VMEM((32, _D), jnp.float32),
            pltpu.SemaphoreType.DMA,
            pltpu.SemaphoreType.DMA,
        ],
    )(_comb_body)
    return f(ys, pos0, pos1)


def kernel(gate_inputs, inputs, Wg, bg, W1, b1, W2, b2):
    wg_pad = jnp.pad(Wg, ((0, 0), (0, 128 - _E)))
    bg_col = jnp.pad(bg, (0, 128 - _E)).reshape(128, 1)
    ri, rw, aux = _route_call(wg_pad, bg_col, gate_inputs)
    te = aux[0, :_NT]
    offs16 = aux[1, :16]
    pp = _pos_call(offs16, ri)
    zi = jnp.zeros((_CHC,), jnp.int32)
    zf = jnp.zeros((_CHC,), jnp.float32)
    tokp0, tokp1, wp0, wp1, pos0, pos1 = _b1_call(pp, rw, zi, zf)
    xs, ws = _b2_call(tokp0, tokp1, wp0, wp1, inputs)
    ys = _ffn_call(te, xs, W1, b1.reshape(_E, 1, _FF), W2,
                   b2.reshape(_E, 1, _D), ws.reshape(_GP, 1))
    return _comb_call(ys, pos0, pos1)


# speedup vs baseline: 1.8808x; 1.0396x over previous
import functools

import jax
import jax.numpy as jnp
from jax import lax
from jax.experimental import pallas as pl
from jax.experimental.pallas import tpu as pltpu
from jax.experimental.pallas import tpu_sc as plsc

_E = 8
_D = 1024
_FF = 4096
_T = 8192
_TM = 256
_G = 2 * _T
_GP = _G + _E * _TM
_NT = _GP // _TM
_NBLK = _T // _TM
_NW = 32
_CHW = _GP // _NW
_CHC = _GP // 16


def _route_body(wg_ref, bg_ref, x_ref, ri_ref, rw_ref, aux_ref, carry_ref):
    b = pl.program_id(0)
    lt = lax.dot_general(wg_ref[...], x_ref[...], (((0,), (1,)), ((), ())),
                         preferred_element_type=jnp.float32)
    lt = lt + bg_ref[...]
    eio = lax.broadcasted_iota(jnp.int32, (128, _TM), 0)
    valid = eio < _E
    neg = jnp.float32(-1e30)
    l0 = jnp.where(valid, lt, neg)
    m0 = jnp.max(l0, axis=0, keepdims=True)
    e0 = jnp.min(jnp.where((l0 == m0) & valid, eio, 127), axis=0, keepdims=True)
    l1 = jnp.where(eio == e0, neg, l0)
    m1 = jnp.max(l1, axis=0, keepdims=True)
    e1 = jnp.min(jnp.where((l1 == m1) & valid, eio, 127), axis=0, keepdims=True)
    t = jnp.exp(m1 - m0)
    inv = 1.0 / (1.0 + t)
    w0, w1 = inv, t * inv

    oh0 = (eio == e0)[:_E, :]
    oh1 = (eio == e1)[:_E, :]
    cnt = oh0.astype(jnp.int32) + oh1.astype(jnp.int32)
    incl = cnt
    for s in (1, 2, 4, 8, 16, 32, 64, 128):
        incl = incl + jnp.concatenate(
            [jnp.zeros((_E, s), jnp.int32), incl[:, :-s]], axis=1)
    excl = incl - cnt
    carry = carry_ref[:, 0:1]
    carry = jnp.where(b == 0, jnp.zeros_like(carry), carry)
    tot = excl + carry
    r0 = jnp.sum(jnp.where(oh0, tot, 0), axis=0, keepdims=True)
    r1 = jnp.sum(jnp.where(oh1, tot, 0), axis=0, keepdims=True)
    newc = carry + incl[:, _TM - 1:_TM]
    carry_ref[:, 0:1] = newc

    ri_ref[...] = jnp.concatenate(
        [e0, e1, r0, r1, jnp.zeros((4, _TM), jnp.int32)], axis=0)[None]
    rw_ref[...] = jnp.concatenate(
        [w0, w1, jnp.zeros((6, _TM), jnp.float32)], axis=0)[None]

    @pl.when(b == pl.num_programs(0) - 1)
    def _():
        pc = ((newc + (_TM - 1)) // _TM) * _TM
        cum = pc
        for s in (1, 2, 4):
            cum = cum + jnp.concatenate(
                [jnp.zeros((s, 1), jnp.int32), cum[:-s]], axis=0)
        gi = lax.broadcasted_iota(jnp.int32, (_E, 128), 1) * _TM
        te = jnp.minimum(jnp.sum((gi >= cum).astype(jnp.int32),
                                 axis=0, keepdims=True), _E - 1)
        lio = lax.broadcasted_iota(jnp.int32, (_E, 128), 1)
        sio = lax.broadcasted_iota(jnp.int32, (_E, 128), 0)
        offs = jnp.sum(jnp.where(sio < lio, jnp.broadcast_to(pc, (_E, 128)), 0),
                       axis=0, keepdims=True)
        aux_ref[...] = jnp.concatenate(
            [te, offs, jnp.zeros((6, 128), jnp.int32)], axis=0)


def _route_call(wg_pad, bg_col, gate_inputs):
    return pl.pallas_call(
        _route_body,
        grid=(_NBLK,),
        in_specs=[
            pl.BlockSpec((_D, 128), lambda b: (0, 0)),
            pl.BlockSpec((128, 1), lambda b: (0, 0)),
            pl.BlockSpec((_TM, _D), lambda b: (b, 0)),
        ],
        out_specs=[
            pl.BlockSpec((1, _E, _TM), lambda b: (b, 0, 0)),
            pl.BlockSpec((1, _E, _TM), lambda b: (b, 0, 0)),
            pl.BlockSpec((_E, 128), lambda b: (0, 0)),
        ],
        out_shape=[
            jax.ShapeDtypeStruct((_NBLK, _E, _TM), jnp.int32),
            jax.ShapeDtypeStruct((_NBLK, _E, _TM), jnp.float32),
            jax.ShapeDtypeStruct((_E, 128), jnp.int32),
        ],
        scratch_shapes=[pltpu.VMEM((_E, 128), jnp.int32)],
        compiler_params=pltpu.CompilerParams(
            dimension_semantics=("arbitrary",)),
    )(wg_pad, bg_col, gate_inputs)


def _pos_body(offs_ref, ri_ref, pp_ref):
    e0 = ri_ref[0, 0:1, :]
    e1 = ri_ref[0, 1:2, :]
    p0 = ri_ref[0, 2:3, :]
    p1 = ri_ref[0, 3:4, :]
    for e in range(_E):
        p0 = p0 + jnp.where(e0 == e, offs_ref[e], 0)
        p1 = p1 + jnp.where(e1 == e, offs_ref[e], 0)
    pp_ref[...] = jnp.concatenate(
        [p0, p1, jnp.zeros((6, _TM), jnp.int32)], axis=0)[None]


def _pos_call(offs16, ri):
    grid_spec = pltpu.PrefetchScalarGridSpec(
        num_scalar_prefetch=1,
        grid=(_NBLK,),
        in_specs=[pl.BlockSpec((1, _E, _TM), lambda b, o: (b, 0, 0))],
        out_specs=pl.BlockSpec((1, _E, _TM), lambda b, o: (b, 0, 0)),
    )
    return pl.pallas_call(
        _pos_body,
        grid_spec=grid_spec,
        out_shape=jax.ShapeDtypeStruct((_NBLK, _E, _TM), jnp.int32),
        compiler_params=pltpu.CompilerParams(
            dimension_semantics=("arbitrary",)),
    )(offs16, ri)


def _b1_body(pp_hbm, rw_hbm, zi_hbm, zf_hbm,
             tokp0_hbm, tokp1_hbm, wp0_hbm, wp1_hbm, pos0_hbm, pos1_hbm,
             pp_v, rw_v, pos_st, val_st, w_st,
             tokbuf, wbuf):
    c = lax.axis_index("c")
    s = lax.axis_index("s")
    wid = s * 2 + c
    base = wid * _TM
    pltpu.sync_copy(pp_hbm.at[wid], pp_v)
    pltpu.sync_copy(rw_hbm.at[wid], rw_v)
    pltpu.sync_copy(zi_hbm, tokbuf.at[pl.ds(s * _CHC, _CHC)])
    pltpu.sync_copy(zf_hbm, wbuf.at[pl.ds(s * _CHC, _CHC)])
    for i in range(16):
        sl = pl.ds(i * 16, 16)
        row, col = i // 8, (i % 8) * 16
        csl = pl.ds(col, 16)
        tk = lax.iota(jnp.int32, 16) + (base + i * 16 + 1)
        pos_st[row, csl] = pp_v[0, sl]
        pos_st[2 + row, csl] = pp_v[1, sl]
        val_st[row, csl] = tk
        val_st[2 + row, csl] = tk
        w_st[row, csl] = rw_v[0, sl]
        w_st[2 + row, csl] = rw_v[1, sl]
    plsc.subcore_barrier()
    for j in range(4):
        pltpu.sync_copy(val_st.at[j], tokbuf.at[pos_st.at[j]])
        pltpu.sync_copy(w_st.at[j], wbuf.at[pos_st.at[j]])
    plsc.subcore_barrier()
    csl = pl.ds(s * _CHC, _CHC)

    @pl.when(c == 0)
    def _():
        pltpu.sync_copy(tokbuf.at[csl], tokp0_hbm.at[csl])
        pltpu.sync_copy(wbuf.at[csl], wp0_hbm.at[csl])

    @pl.when(c == 1)
    def _():
        pltpu.sync_copy(tokbuf.at[csl], tokp1_hbm.at[csl])
        pltpu.sync_copy(wbuf.at[csl], wp1_hbm.at[csl])

    pltpu.sync_copy(pp_v.at[0], pos0_hbm.at[pl.ds(base, _TM)])
    pltpu.sync_copy(pp_v.at[1], pos1_hbm.at[pl.ds(base, _TM)])


def _b1_call(pp, rw, zi, zf):
    mesh = plsc.VectorSubcoreMesh(core_axis_name="c", subcore_axis_name="s")
    f = functools.partial(
        pl.kernel, mesh=mesh,
        out_type=[
            jax.ShapeDtypeStruct((_GP,), jnp.int32),
            jax.ShapeDtypeStruct((_GP,), jnp.int32),
            jax.ShapeDtypeStruct((_GP,), jnp.float32),
            jax.ShapeDtypeStruct((_GP,), jnp.float32),
            jax.ShapeDtypeStruct((_T,), jnp.int32),
            jax.ShapeDtypeStruct((_T,), jnp.int32),
        ],
        scratch_types=[
            pltpu.VMEM((_E, _TM), jnp.int32),
            pltpu.VMEM((_E, _TM), jnp.float32),
            pltpu.VMEM((4, 128), jnp.int32),
            pltpu.VMEM((4, 128), jnp.int32),
            pltpu.VMEM((4, 128), jnp.float32),
            pltpu.VMEM_SHARED((_GP,), jnp.int32),
            pltpu.VMEM_SHARED((_GP,), jnp.float32),
        ],
    )(_b1_body)
    return f(pp, rw, zi, zf)


def _b2_body(tokp0_hbm, tokp1_hbm, wp0_hbm, wp1_hbm, x_hbm, xs_hbm, ws_hbm,
             ta_v, tb_v, tok_v, wa_v, wb_v, wc_v, rows_v, rows2_v,
             sem, sem2, wsem0, wsem1):
    c = lax.axis_index("c")
    s = lax.axis_index("s")
    wid = s * 2 + c
    base = wid * _CHW
    sl = pl.ds(base, _CHW)
    pltpu.sync_copy(tokp0_hbm.at[sl], ta_v)
    pltpu.sync_copy(tokp1_hbm.at[sl], tb_v)
    pltpu.sync_copy(wp0_hbm.at[sl], wa_v)
    pltpu.sync_copy(wp1_hbm.at[sl], wb_v)
    for j in range(_CHW // 16):
        vs = pl.ds(j * 16, 16)
        tok_v[vs] = jnp.maximum(ta_v[vs] + tb_v[vs] - 1, 0)
        wc_v[vs] = wa_v[vs] + wb_v[vs]
    pltpu.sync_copy(wc_v, ws_hbm.at[sl])
    nch = _CHW // 48
    rows = (rows_v, rows2_v)
    gsem = (sem, sem2)
    wsem = (wsem0, wsem1)
    gh = [None, None]
    wh = [None, None]
    for k in range(nch):
        b = k & 1
        if wh[b] is not None:
            wh[b].wait()
        gh[b] = pltpu.async_copy(x_hbm.at[tok_v.at[pl.ds(k * 48, 48)]],
                                 rows[b], gsem[b])
        if k > 0:
            p = 1 - b
            gh[p].wait()
            wh[p] = pltpu.async_copy(
                rows[p], xs_hbm.at[pl.ds(base + (k - 1) * 48, 48)], wsem[p])
    b = (nch - 1) & 1
    gh[b].wait()
    wh[b] = pltpu.async_copy(
        rows[b], xs_hbm.at[pl.ds(base + (nch - 1) * 48, 48)], wsem[b])
    wh[0].wait()
    wh[1].wait()


def _b2_call(tokp0, tokp1, wp0, wp1, x):
    mesh = plsc.VectorSubcoreMesh(core_axis_name="c", subcore_axis_name="s")
    f = functools.partial(
        pl.kernel, mesh=mesh,
        out_type=[
            jax.ShapeDtypeStruct((_GP, _D), jnp.float32),
            jax.ShapeDtypeStruct((_GP,), jnp.float32),
        ],
        scratch_types=[
            pltpu.VMEM((_CHW,), jnp.int32),
            pltpu.VMEM((_CHW,), jnp.int32),
            pltpu.VMEM((_CHW,), jnp.int32),
            pltpu.VMEM((_CHW,), jnp.float32),
            pltpu.VMEM((_CHW,), jnp.float32),
            pltpu.VMEM((_CHW,), jnp.float32),
            pltpu.VMEM((48, _D), jnp.float32),
            pltpu.VMEM((48, _D), jnp.float32),
            pltpu.SemaphoreType.DMA,
            pltpu.SemaphoreType.DMA,
            pltpu.SemaphoreType.DMA,
            pltpu.SemaphoreType.DMA,
        ],
    )(_b2_body)
    return f(tokp0, tokp1, wp0, wp1, x)


def _ffn_body(te_ref, x_ref, w1_ref, b1_ref, w2_ref, b2_ref, ws_ref, y_ref):
    h = jnp.dot(x_ref[...], w1_ref[0], preferred_element_type=jnp.float32)
    h = h + b1_ref[0]
    h = h * (1.0 / (1.0 + jnp.exp(-h)))
    y = jnp.dot(h, w2_ref[0], preferred_element_type=jnp.float32)
    y = y + b2_ref[0]
    y_ref[...] = y * ws_ref[...]


def _ffn_call(te, xs, w1, b1, w2, b2, ws):
    grid_spec = pltpu.PrefetchScalarGridSpec(
        num_scalar_prefetch=1,
        grid=(_NT,),
        in_specs=[
            pl.BlockSpec((_TM, _D), lambda g, te: (g, 0)),
            pl.BlockSpec((1, _D, _FF), lambda g, te: (te[g], 0, 0),
                         pipeline_mode=pl.Buffered(buffer_count=2)),
            pl.BlockSpec((1, 1, _FF), lambda g, te: (te[g], 0, 0)),
            pl.BlockSpec((1, _FF, _D), lambda g, te: (te[g], 0, 0),
                         pipeline_mode=pl.Buffered(buffer_count=1)),
            pl.BlockSpec((1, 1, _D), lambda g, te: (te[g], 0, 0)),
            pl.BlockSpec((_TM, 1), lambda g, te: (g, 0)),
        ],
        out_specs=pl.BlockSpec((_TM, _D), lambda g, te: (g, 0)),
    )
    return pl.pallas_call(
        _ffn_body,
        grid_spec=grid_spec,
        out_shape=jax.ShapeDtypeStruct((_GP, _D), jnp.float32),
        compiler_params=pltpu.CompilerParams(
            dimension_semantics=("arbitrary",),
            vmem_limit_bytes=128 * 1024 * 1024),
    )(te, xs, w1, b1, w2, b2, ws)


def _comb_body(y_hbm, pos0_hbm, pos1_hbm, out_hbm, p0_v, p1_v, a_v, b_v,
               sem0, sem1):
    c = lax.axis_index("c")
    s = lax.axis_index("s")
    wid = s * 2 + c
    base = wid * _TM
    pltpu.sync_copy(pos0_hbm.at[pl.ds(base, _TM)], p0_v)
    pltpu.sync_copy(pos1_hbm.at[pl.ds(base, _TM)], p1_v)
    for m in range(_TM // 32):
        ca = pltpu.async_copy(y_hbm.at[p0_v.at[pl.ds(m * 32, 32)]], a_v, sem0)
        cb = pltpu.async_copy(y_hbm.at[p1_v.at[pl.ds(m * 32, 32)]], b_v, sem1)
        ca.wait()
        cb.wait()

        def _add_row(r, carry):
            for cc in range(_D // 16):
                vs = pl.ds(cc * 16, 16)
                a_v[r, vs] = a_v[r, vs] + b_v[r, vs]
            return carry

        lax.fori_loop(0, 32, _add_row, 0)
        pltpu.sync_copy(a_v, out_hbm.at[pl.ds(base + m * 32, 32)])


def _comb_call(ys, pos0, pos1):
    mesh = plsc.VectorSubcoreMesh(core_axis_name="c", subcore_axis_name="s")
    f = functools.partial(
        pl.kernel, mesh=mesh,
        out_type=jax.ShapeDtypeStruct((_T, _D), jnp.float32),
        scratch_types=[
            pltpu.VMEM((_TM,), jnp.int32),
            pltpu.VMEM((_TM,), jnp.int32),
            pltpu.VMEM((32, _D), jnp.float32),
            pltpu.VMEM((32, _D), jnp.float32),
            pltpu.SemaphoreType.DMA,
            pltpu.SemaphoreType.DMA,
        ],
    )(_comb_body)
    return f(ys, pos0, pos1)


def kernel(gate_inputs, inputs, Wg, bg, W1, b1, W2, b2):
    wg_pad = jnp.pad(Wg, ((0, 0), (0, 128 - _E)))
    bg_col = jnp.pad(bg, (0, 128 - _E)).reshape(128, 1)
    ri, rw, aux = _route_call(wg_pad, bg_col, gate_inputs)
    te = aux[0, :_NT]
    offs16 = aux[1, :16]
    pp = _pos_call(offs16, ri)
    zi = jnp.zeros((_CHC,), jnp.int32)
    zf = jnp.zeros((_CHC,), jnp.float32)
    tokp0, tokp1, wp0, wp1, pos0, pos1 = _b1_call(pp, rw, zi, zf)
    xs, ws = _b2_call(tokp0, tokp1, wp0, wp1, inputs)
    ys = _ffn_call(te, xs, W1, b1.reshape(_E, 1, _FF), W2,
                   b2.reshape(_E, 1, _D), ws.reshape(_GP, 1))
    return _comb_call(ys, pos0, pos1)
